# Initial kernel scaffold; baseline (speedup 1.0000x reference)
import jax
import jax.numpy as jnp
from jax.experimental import pallas as pl


def kernel(X, pos_table):
    raise NotImplementedError("write your pallas kernel here")


# speedup vs baseline: 1.4874x; 1.4874x over previous
import jax
import jax.numpy as jnp
from jax.experimental import pallas as pl


_BLOCK_T = 512


def _add_kernel(x_ref, pos_ref, o_ref):
    o_ref[...] = x_ref[...] + pos_ref[...]


def kernel(X, pos_table):
    N, T, D = X.shape
    bt = min(_BLOCK_T, T)
    grid = (T // bt, N)
    return pl.pallas_call(
        _add_kernel,
        grid=grid,
        in_specs=[
            pl.BlockSpec((1, bt, D), lambda t, n: (n, t, 0)),
            pl.BlockSpec((bt, D), lambda t, n: (t, 0)),
        ],
        out_specs=pl.BlockSpec((1, bt, D), lambda t, n: (n, t, 0)),
        out_shape=jax.ShapeDtypeStruct((N, T, D), X.dtype),
    )(X, pos_table)


# speedup vs baseline: 1.6656x; 1.1198x over previous
import jax
import jax.numpy as jnp
from jax.experimental import pallas as pl


_BLOCK_T = 1024


def _add_kernel(x_ref, pos_ref, o_ref):
    o_ref[...] = x_ref[...] + pos_ref[...]


def kernel(X, pos_table):
    N, T, D = X.shape
    bt = min(_BLOCK_T, T)
    grid = (T // bt, N)
    return pl.pallas_call(
        _add_kernel,
        grid=grid,
        in_specs=[
            pl.BlockSpec((1, bt, D), lambda t, n: (n, t, 0)),
            pl.BlockSpec((bt, D), lambda t, n: (t, 0)),
        ],
        out_specs=pl.BlockSpec((1, bt, D), lambda t, n: (n, t, 0)),
        out_shape=jax.ShapeDtypeStruct((N, T, D), X.dtype),
    )(X, pos_table)


# speedup vs baseline: 1.7337x; 1.0409x over previous
import jax
import jax.numpy as jnp
from jax.experimental import pallas as pl


_BLOCK_T = 2048


def _add_kernel(x_ref, pos_ref, o_ref):
    o_ref[...] = x_ref[...] + pos_ref[...]


def kernel(X, pos_table):
    N, T, D = X.shape
    bt = min(_BLOCK_T, T)
    grid = (T // bt, N)
    return pl.pallas_call(
        _add_kernel,
        grid=grid,
        in_specs=[
            pl.BlockSpec((1, bt, D), lambda t, n: (n, t, 0)),
            pl.BlockSpec((bt, D), lambda t, n: (t, 0)),
        ],
        out_specs=pl.BlockSpec((1, bt, D), lambda t, n: (n, t, 0)),
        out_shape=jax.ShapeDtypeStruct((N, T, D), X.dtype),
    )(X, pos_table)
